# baseline (speedup 1.0000x reference)
import jax
import jax.numpy as jnp
from jax import lax
from jax.experimental import pallas as pl
from jax.experimental.pallas import tpu as pltpu
from jax.experimental.pallas import tpu_sc as plsc

B = 16384
NC, NS = 2, 16
NW = NC * NS
BPW = B // NW
CHUNK = 128
NCHUNK = BPW // CHUNK

D_S, D_M, D_E = 6, 54, 6
D_OUT = D_S + D_M + D_E
P_S, P_M, P_E = 8, 56, 8
RB = 8

_mesh = plsc.VectorSubcoreMesh(
    core_axis_name="c", subcore_axis_name="s", num_cores=NC, num_subcores=NS
)

_MAGIC = {6: 10923, 54: 1214, 7: 9363}


def _patterns(iota, width, col_off):
    out = []
    for v in range((RB * width) // 16):
        q = v * 16 + iota
        b = (q * _MAGIC[width]) >> 16
        j = q - b * width
        d = b * D_OUT + col_off + j
        out.append((b, j, d))
    return out


_SCRATCH = [
    pltpu.VMEM((9, BPW), jnp.int32),
    pltpu.VMEM((CHUNK, D_S), jnp.float32),
    pltpu.VMEM((CHUNK, D_S), jnp.float32),
    pltpu.VMEM((CHUNK, D_M), jnp.float32),
    pltpu.VMEM((CHUNK, D_M), jnp.float32),
    pltpu.VMEM((CHUNK, D_E), jnp.float32),
    pltpu.VMEM((CHUNK, D_E), jnp.float32),
    pltpu.VMEM((BPW * D_OUT,), jnp.float32),
    pltpu.SemaphoreType.DMA,
]


def _se3_body(s_hbm, m_hbm, e_hbm, ix_hbm, out_hbm,
              ix_v, s0_v, s1_v, m0_v, m1_v, e0_v, e1_v, out_v, sem):
    wid = lax.axis_index("s") * NC + lax.axis_index("c")
    base = wid * BPW

    pltpu.sync_copy(ix_hbm.at[:, pl.ds(base, BPW)], ix_v)

    iota = lax.iota(jnp.int32, 16)
    tabs = [(s_hbm, 0, 1, 2, s0_v, s1_v, D_S, P_S, 0),
            (m_hbm, 3, 4, 5, m0_v, m1_v, D_M, P_M, D_S),
            (e_hbm, 6, 7, 8, e0_v, e1_v, D_E, P_E, D_S + D_M)]

    for c in range(NCHUNK):
        cols = pl.ds(c * CHUNK, CHUNK)
        copies = []
        for tab, r0, r1, _ro, w0, w1, _w, _p, _co in tabs:
            copies.append(pltpu.async_copy(tab.at[ix_v.at[r0, cols]], w0, sem))
            copies.append(pltpu.async_copy(tab.at[ix_v.at[r1, cols]], w1, sem))
        for cp in copies:
            cp.wait()

        def body(blk, carry, _c=c):
            b0 = blk * RB
            brow = _c * CHUNK + b0
            for _tab, _r0, _r1, ro, w0, w1, w, p, co in tabs:
                for bc, jc, dc in _patterns(iota, w, co):
                    bl = b0 + bc
                    o = plsc.load_gather(ix_v, [jnp.full((16,), ro, jnp.int32),
                                                _c * CHUNK + bl])
                    cw0 = o + jc
                    inw0 = cw0 < w
                    c0 = jnp.where(inw0, cw0, w - 1)
                    c1 = jnp.where(inw0, 0, cw0 - w)
                    x0 = bl * w + c0
                    x1 = bl * w + c1
                    if p == 8:
                        r0c, c0c = x0 >> 3, x0 & 7
                        r1c, c1c = x1 >> 3, x1 & 7
                    else:
                        r0c = ((x0 >> 3) * _MAGIC[7]) >> 16
                        c0c = x0 - r0c * p
                        r1c = ((x1 >> 3) * _MAGIC[7]) >> 16
                        c1c = x1 - r1c * p
                    v0 = plsc.load_gather(w0, [r0c, c0c])
                    v1 = plsc.load_gather(w1, [r1c, c1c])
                    vec = jnp.where(inw0, v0, v1)
                    plsc.store_scatter(out_v, [brow * D_OUT + dc], vec)
            return carry

        lax.fori_loop(0, CHUNK // RB, body, 0)

    pltpu.sync_copy(out_v, out_hbm.at[pl.ds(base * D_OUT, BPW * D_OUT)])


_se3_lookup = pl.kernel(
    _se3_body,
    out_type=jax.ShapeDtypeStruct((B * D_OUT,), jnp.float32),
    mesh=_mesh,
    compiler_params=pltpu.CompilerParams(
        use_tc_tiling_on_sc=False, needs_layout_passes=False),
    scratch_types=_SCRATCH,
)


def _windows(idx, n, w, p):
    r = idx.astype(jnp.int32)
    k0 = (p * r) // w
    o = p * r - w * k0
    kmax = (n * p - w) // w
    k1 = jnp.minimum(k0 + 1, kmax)
    return k0, k1, o


def kernel(start_table, mid_table, end_table, idx_start, idx_mid, idx_end):
    ks0, ks1, os_ = _windows(idx_start, start_table.shape[0], D_S, P_S)
    km0, km1, om = _windows(idx_mid, mid_table.shape[0], D_M, P_M)
    ke0, ke1, oe = _windows(idx_end, end_table.shape[0], D_E, P_E)
    ix = jnp.stack([ks0, ks1, os_, km0, km1, om, ke0, ke1, oe])
    flat = _se3_lookup(start_table, mid_table, end_table, ix)
    return flat.reshape(B, D_OUT)


# speedup vs baseline: 1.2515x; 1.2515x over previous
import jax
import jax.numpy as jnp
from jax import lax
from jax.experimental import pallas as pl
from jax.experimental.pallas import tpu as pltpu
from jax.experimental.pallas import tpu_sc as plsc

B = 16384
NC, NS = 2, 16
NW = NC * NS
BPW = B // NW
CH = 64
NCH = BPW // CH

D_S, D_M, D_E = 6, 54, 6
D_OUT = D_S + D_M + D_E
WIN = 128
RB = 8

_mesh = plsc.VectorSubcoreMesh(
    core_axis_name="c", subcore_axis_name="s", num_cores=NC, num_subcores=NS
)

_MAGIC = {6: 10923, 54: 1214}


def _patterns(iota, width, col_off):
    out = []
    for v in range((RB * width) // 16):
        q = v * 16 + iota
        b = (q * _MAGIC[width]) >> 16
        j = q - b * width
        d = b * D_OUT + col_off + j
        out.append((b, j, d))
    return out


_SCRATCH = [
    pltpu.VMEM((9, BPW), jnp.int32),
    pltpu.VMEM((CH, WIN), jnp.float32),
    pltpu.VMEM((CH, WIN), jnp.float32),
    pltpu.VMEM((CH, WIN), jnp.float32),
    pltpu.VMEM((CH, WIN), jnp.float32),
    pltpu.VMEM((CH, WIN), jnp.float32),
    pltpu.VMEM((CH, WIN), jnp.float32),
    pltpu.VMEM((BPW * D_OUT,), jnp.float32),
    pltpu.SemaphoreType.DMA,
]


def _se3_body(s_hbm, m_hbm, e_hbm, ix_hbm, out_hbm,
              ix_v, s0_v, s1_v, m0_v, m1_v, e0_v, e1_v, out_v, sem):
    wid = lax.axis_index("s") * NC + lax.axis_index("c")
    base = wid * BPW

    pltpu.sync_copy(ix_hbm.at[:, pl.ds(base, BPW)], ix_v)

    iota = lax.iota(jnp.int32, 16)
    tabs = [(s_hbm, 0, 1, 2, s0_v, s1_v, D_S, 0),
            (m_hbm, 3, 4, 5, m0_v, m1_v, D_M, D_S),
            (e_hbm, 6, 7, 8, e0_v, e1_v, D_E, D_S + D_M)]

    for c in range(NCH):
        cols = pl.ds(c * CH, CH)
        copies = []
        for tab, r0, r1, _ro, w0, w1, _w, _co in tabs:
            copies.append(pltpu.async_copy(tab.at[ix_v.at[r0, cols]], w0, sem))
            copies.append(pltpu.async_copy(tab.at[ix_v.at[r1, cols]], w1, sem))
        for cp in copies:
            cp.wait()

        def body(blk, carry, _c=c):
            b0 = blk * RB
            drow = _c * CH + b0
            for _tab, _r0, _r1, ro, w0, w1, w, co in tabs:
                ro_vec = jnp.full((16,), ro, jnp.int32)
                for bc, jc, dc in _patterns(iota, w, co):
                    bl = b0 + bc
                    o = plsc.load_gather(ix_v, [ro_vec, _c * CH + bl])
                    col = o + jc
                    inw0 = col < WIN
                    c0 = jnp.where(inw0, col, WIN - 1)
                    c1 = jnp.where(inw0, 0, col - WIN)
                    v0 = plsc.load_gather(w0, [bl, c0])
                    v1 = plsc.load_gather(w1, [bl, c1])
                    vec = jnp.where(inw0, v0, v1)
                    plsc.store_scatter(out_v, [drow * D_OUT + dc], vec)
            return carry

        lax.fori_loop(0, CH // RB, body, 0)

    pltpu.sync_copy(out_v, out_hbm.at[pl.ds(base * D_OUT, BPW * D_OUT)])


_se3_lookup = pl.kernel(
    _se3_body,
    out_type=jax.ShapeDtypeStruct((B * D_OUT,), jnp.float32),
    mesh=_mesh,
    compiler_params=pltpu.CompilerParams(
        use_tc_tiling_on_sc=False, needs_layout_passes=False),
    scratch_types=_SCRATCH,
)


def _windows_view(t):
    flat = t.reshape(-1)
    n = flat.shape[0]
    n128 = -(-n // WIN) * WIN
    if n128 != n:
        flat = jnp.pad(flat, (0, n128 - n))
    return flat.reshape(n128 // WIN, WIN)


def _windows_idx(idx, w, nrows):
    r = idx.astype(jnp.int32)
    k0 = (w * r) >> 7
    o = (w * r) & (WIN - 1)
    k1 = jnp.minimum(k0 + 1, nrows - 1)
    return k0, k1, o


def kernel(start_table, mid_table, end_table, idx_start, idx_mid, idx_end):
    sv = _windows_view(start_table)
    mv = _windows_view(mid_table)
    ev = _windows_view(end_table)
    ks0, ks1, os_ = _windows_idx(idx_start, D_S, sv.shape[0])
    km0, km1, om = _windows_idx(idx_mid, D_M, mv.shape[0])
    ke0, ke1, oe = _windows_idx(idx_end, D_E, ev.shape[0])
    ix = jnp.stack([ks0, ks1, os_, km0, km1, om, ke0, ke1, oe])
    flat = _se3_lookup(sv, mv, ev, ix)
    return flat.reshape(B, D_OUT)


# speedup vs baseline: 1.5008x; 1.1992x over previous
import jax
import jax.numpy as jnp
from jax import lax
from jax.experimental import pallas as pl
from jax.experimental.pallas import tpu as pltpu
from jax.experimental.pallas import tpu_sc as plsc

B = 16384
NC, NS = 2, 16
NW = NC * NS
BPW = B // NW
CHUNK = 128
NCHUNK = BPW // CHUNK

D_S, D_M, D_E = 6, 54, 6
D_OUT = D_S + D_M + D_E
P_S, P_M, P_E = 8, 56, 8
RB = 8

_mesh = plsc.VectorSubcoreMesh(
    core_axis_name="c", subcore_axis_name="s", num_cores=NC, num_subcores=NS
)

_MAGIC = {6: 10923, 54: 1214}


def _patterns(iota, width, col_off):
    out = []
    for v in range((RB * width) // 16):
        q = v * 16 + iota
        b = (q * _MAGIC[width]) >> 16
        j = q - b * width
        d = b * D_OUT + col_off + j
        out.append((b, j, d))
    return out


_SCRATCH = [
    pltpu.VMEM((NCHUNK, CHUNK), jnp.int32),
    pltpu.VMEM((NCHUNK, CHUNK), jnp.int32),
    pltpu.VMEM((NCHUNK, CHUNK), jnp.int32),
    pltpu.VMEM((BPW, P_S), jnp.float32),
    pltpu.VMEM((BPW, P_M), jnp.float32),
    pltpu.VMEM((BPW, P_E), jnp.float32),
    pltpu.VMEM((BPW * D_OUT,), jnp.float32),
    pltpu.SemaphoreType.DMA,
]


def _se3_body(s_hbm, m_hbm, e_hbm, is_hbm, im_hbm, ie_hbm, out_hbm,
              is_v, im_v, ie_v, s_v, m_v, e_v, out_v, sem):
    wid = lax.axis_index("s") * NC + lax.axis_index("c")
    base = wid * BPW

    for j in range(NCHUNK):
        off = base + j * CHUNK
        pltpu.sync_copy(is_hbm.at[pl.ds(off, CHUNK)], is_v.at[j])
        pltpu.sync_copy(im_hbm.at[pl.ds(off, CHUNK)], im_v.at[j])
        pltpu.sync_copy(ie_hbm.at[pl.ds(off, CHUNK)], ie_v.at[j])

    copies = []
    for j in range(NCHUNK):
        rows = pl.ds(j * CHUNK, CHUNK)
        copies.append(pltpu.async_copy(s_hbm.at[is_v.at[j]], s_v.at[rows], sem))
        copies.append(pltpu.async_copy(m_hbm.at[im_v.at[j]], m_v.at[rows], sem))
        copies.append(pltpu.async_copy(e_hbm.at[ie_v.at[j]], e_v.at[rows], sem))
    for c in copies:
        c.wait()

    iota = lax.iota(jnp.int32, 16)
    tables = [(s_v, _patterns(iota, D_S, 0)),
              (m_v, _patterns(iota, D_M, D_S)),
              (e_v, _patterns(iota, D_E, D_S + D_M))]

    def body(blk, carry):
        b0 = blk * RB
        d0 = b0 * D_OUT
        for src_v, pats in tables:
            for bc, jc, dc in pats:
                vec = plsc.load_gather(src_v, [b0 + bc, jc])
                plsc.store_scatter(out_v, [d0 + dc], vec)
        return carry

    lax.fori_loop(0, BPW // RB, body, 0)

    pltpu.sync_copy(out_v, out_hbm.at[pl.ds(base * D_OUT, BPW * D_OUT)])


_se3_lookup = pl.kernel(
    _se3_body,
    out_type=jax.ShapeDtypeStruct((B * D_OUT,), jnp.float32),
    mesh=_mesh,
    compiler_params=pltpu.CompilerParams(
        use_tc_tiling_on_sc=False, needs_layout_passes=False),
    scratch_types=_SCRATCH,
)


def _widen(t, extra):
    return jnp.concatenate([t, t[:, :extra]], axis=1)


def kernel(start_table, mid_table, end_table, idx_start, idx_mid, idx_end):
    flat = _se3_lookup(
        _widen(start_table, P_S - D_S),
        _widen(mid_table, P_M - D_M),
        _widen(end_table, P_E - D_E),
        idx_start.astype(jnp.int32), idx_mid.astype(jnp.int32),
        idx_end.astype(jnp.int32),
    )
    return flat.reshape(B, D_OUT)
